# baseline (speedup 1.0000x reference)
import functools
import jax
import jax.numpy as jnp
from jax.experimental import pallas as pl
from jax.experimental.pallas import tpu as pltpu
from jax.experimental.pallas import tpu_sc as plsc

_L = 16


def kernel(x, table):
    B, T = x.shape
    V, D = table.shape
    NB = B // 128
    xT = x.T
    tableT = table.T
    mesh = plsc.VectorSubcoreMesh(core_axis_name="core",
                                  subcore_axis_name="subcore")

    @functools.partial(
        pl.kernel,
        out_type=jax.ShapeDtypeStruct((T, D // 8, NB, 8, 128), jnp.float32),
        mesh=mesh,
        scratch_types=[pltpu.VMEM((D * V,), jnp.float32)],
        compiler_params=pltpu.CompilerParams(use_tc_tiling_on_sc=False,
                                             needs_layout_passes=False),
    )
    def emb(tableT_hbm, xT_hbm, o_hbm, tab_vmem):
        pltpu.sync_copy(tableT_hbm, tab_vmem)

        def body(i_vmem, o_vmem):
            for g in range(128 // _L):
                idx16 = i_vmem[0, pl.ds(g * _L, _L)]
                for d in range(D):
                    vals = plsc.load_gather(tab_vmem, [idx16 + (d * V)])
                    o_vmem[0, d // 8, 0, d % 8, pl.ds(g * _L, _L)] = vals

        pltpu.emit_pipeline(
            body,
            grid=(T, NB),
            in_specs=[pl.BlockSpec((1, 128), index_map=lambda t, b: (t, b))],
            out_specs=[pl.BlockSpec((1, D // 8, 1, 8, 128),
                                    index_map=lambda t, b: (t, 0, b, 0, 0))],
            core_axis_name=("core", "subcore"),
            dimension_semantics=(pltpu.PARALLEL, pltpu.PARALLEL),
        )(xT_hbm, o_hbm)

    out5 = emb(tableT.reshape(-1), xT)
    return jnp.transpose(out5, (2, 4, 0, 1, 3)).reshape(B, T, D)


# speedup vs baseline: 1.3775x; 1.3775x over previous
import functools
import jax
import jax.numpy as jnp
from jax.experimental import pallas as pl
from jax.experimental.pallas import tpu as pltpu
from jax.experimental.pallas import tpu_sc as plsc

_L = 16


def kernel(x, table):
    B, T = x.shape
    V, D = table.shape
    NB = B // 128
    xT = x.T
    tableT = table.T
    mesh = plsc.VectorSubcoreMesh(core_axis_name="core",
                                  subcore_axis_name="subcore")

    @functools.partial(
        pl.kernel,
        out_type=jax.ShapeDtypeStruct((T, D // 8, NB, 8, 128), jnp.float32),
        mesh=mesh,
        scratch_types=[pltpu.VMEM((D * V,), jnp.float32)],
        compiler_params=pltpu.CompilerParams(use_tc_tiling_on_sc=False,
                                             needs_layout_passes=False),
    )
    def emb(tableT_hbm, xT_hbm, o_hbm, tab_vmem):
        pltpu.sync_copy(tableT_hbm, tab_vmem)

        def body(i_vmem, o_vmem):
            @pl.loop(0, 128 // _L)
            def _(g):
                idx16 = i_vmem[0, pl.ds(g * _L, _L)]
                for d in range(D):
                    vals = plsc.load_gather(tab_vmem, [idx16 + (d * V)])
                    o_vmem[0, d // 8, 0, d % 8, pl.ds(g * _L, _L)] = vals

        pltpu.emit_pipeline(
            body,
            grid=(T, NB),
            in_specs=[pl.BlockSpec((1, 128), index_map=lambda t, b: (t, b))],
            out_specs=[pl.BlockSpec((1, D // 8, 1, 8, 128),
                                    index_map=lambda t, b: (t, 0, b, 0, 0))],
            core_axis_name=("core", "subcore"),
            dimension_semantics=(pltpu.PARALLEL, pltpu.PARALLEL),
        )(xT_hbm, o_hbm)

    out5 = emb(tableT.reshape(-1), xT)
    return jnp.transpose(out5, (2, 4, 0, 1, 3)).reshape(B, T, D)
